# baseline (speedup 1.0000x reference)
import functools
import math

import jax
import jax.numpy as jnp
from jax import lax
from jax.experimental import pallas as pl
from jax.experimental.pallas import tpu as pltpu
from jax.experimental.pallas import tpu_sc as plsc


def _pos_encoding(seq_len, dim):
    position = jnp.arange(0, seq_len, dtype=jnp.float32)[:, None]
    div_term = jnp.exp(
        jnp.arange(0, dim, 2, dtype=jnp.float32) * -(math.log(10000.0) / dim)
    )
    pe = jnp.zeros((seq_len, dim), dtype=jnp.float32)
    pe = pe.at[:, 0::2].set(jnp.sin(position * div_term))
    pe = pe.at[:, 1::2].set(jnp.cos(position * div_term))
    return pe


NC, NS = 2, 16
NW = NC * NS
L = 16


@jax.jit
def _sc_stage(table_t, tailp):
    dim, vocab = table_t.shape
    n_full = vocab // 128
    rem = vocab - n_full * 128
    per_w = n_full // NW
    extra = n_full - per_w * NW

    mesh = plsc.VectorSubcoreMesh(core_axis_name="c", subcore_axis_name="s")

    @functools.partial(
        pl.kernel,
        mesh=mesh,
        out_type=jax.ShapeDtypeStruct((vocab, 128), jnp.float32),
        scratch_types=[
            pltpu.VMEM((dim, 128), jnp.float32),
            pltpu.VMEM((dim, 128), jnp.float32),
            pltpu.VMEM((128, 128), jnp.float32),
            pltpu.VMEM((128, 128), jnp.float32),
            pltpu.SemaphoreType.DMA,
            pltpu.SemaphoreType.DMA,
            pltpu.SemaphoreType.DMA,
            pltpu.SemaphoreType.DMA,
        ],
        compiler_params=pltpu.CompilerParams(
            use_tc_tiling_on_sc=True, needs_layout_passes=False),
    )
    def body(tt_hbm, tail_hbm, out_hbm, in0, in1, wb0, wb1, gi0, gi1, go0, go1):
        wid = lax.axis_index("s") * NC + lax.axis_index("c")
        iota = lax.iota(jnp.int32, L)
        bufs = ((in0, wb0, gi0, go0), (in1, wb1, gi1, go1))

        def cstart(t):
            blk = jnp.where(t < per_w, wid * per_w + t, n_full - extra + wid)
            return blk * 128

        def issue_in(t, inb, gisem):
            pltpu.async_copy(
                tt_hbm.at[:, pl.ds(cstart(t), 128)], inb, gisem)

        my_blocks = jnp.where(wid < extra, per_w + 1, per_w)

        for b in range(2):
            issue_in(b, bufs[b][0], bufs[b][2])

        def process(t, buf):
            inb, wbb, gisem, gosem = buf
            pltpu.make_async_copy(
                tt_hbm.at[:, pl.ds(0, 128)], inb, gisem).wait()

            @pl.when(t >= 2)
            def _():
                pltpu.make_async_copy(
                    wbb, out_hbm.at[pl.ds(0, 128)], gosem).wait()

            @plsc.parallel_loop(0, 128, 1, unroll=2)
            def _(w):
                colw = jnp.full((L,), w, jnp.int32)
                for dg in range(dim // L):
                    v = plsc.load_gather(inb, (iota + dg * L, colw))
                    wbb[w, pl.ds(dg * L, L)] = v

            pltpu.async_copy(
                wbb, out_hbm.at[pl.ds(cstart(t), 128)], gosem)

            @pl.when(t + 2 < my_blocks)
            def _():
                issue_in(t + 2, inb, gisem)

        def step2(g, carry):
            process(2 * g, bufs[0])
            process(2 * g + 1, bufs[1])
            return carry

        lax.fori_loop(0, per_w // 2, step2, 0)

        @pl.when(wid < extra)
        def _():
            process(per_w, bufs[0])

        for b in range(2):
            _, wbb, _, gosem = bufs[b]
            pltpu.make_async_copy(
                wbb, out_hbm.at[pl.ds(0, 128)], gosem).wait()

        @pl.when(wid == NW - 1)
        def _():
            _, wbb, _, gosem = bufs[0]
            pltpu.sync_copy(tail_hbm, wbb.at[pl.ds(0, rem)])
            pltpu.async_copy(
                wbb.at[pl.ds(0, rem)],
                out_hbm.at[pl.ds(n_full * 128, rem)], gosem)
            pltpu.make_async_copy(
                wbb.at[pl.ds(0, rem)],
                out_hbm.at[pl.ds(0, rem)], gosem).wait()

    return body(table_t, tailp)


@functools.partial(jax.jit, static_argnums=(3, 4))
def _sc_embed(idx, pe, staged, batch, seq):
    n_rows = batch * seq
    dim = 64
    n_chunks = batch // NW
    s_a = 128
    s_b = seq - s_a

    mesh = plsc.VectorSubcoreMesh(core_axis_name="c", subcore_axis_name="s")

    @functools.partial(
        pl.kernel,
        mesh=mesh,
        out_type=jax.ShapeDtypeStruct((n_rows, dim), jnp.float32),
        scratch_types=[
            pltpu.VMEM((seq, dim), jnp.float32),
            pltpu.VMEM((seq,), jnp.int32),
            pltpu.VMEM((seq,), jnp.int32),
            pltpu.VMEM((seq, 128), jnp.float32),
            pltpu.VMEM((seq, 128), jnp.float32),
            pltpu.VMEM((seq, dim), jnp.float32),
            pltpu.VMEM((seq, dim), jnp.float32),
            pltpu.SemaphoreType.DMA,
            pltpu.SemaphoreType.DMA,
            pltpu.SemaphoreType.DMA,
            pltpu.SemaphoreType.DMA,
            pltpu.SemaphoreType.DMA,
            pltpu.SemaphoreType.DMA,
        ],
        compiler_params=pltpu.CompilerParams(use_tc_tiling_on_sc=False),
    )
    def body(idx_hbm, pe_hbm, table_hbm, out_hbm,
             pe_v, idx0, idx1, rows0, rows1, outs0, outs1,
             gs0, gs1, is0, is1, os0, os1):
        wid = lax.axis_index("s") * NC + lax.axis_index("c")
        first = wid * n_chunks
        pltpu.sync_copy(pe_hbm, pe_v)

        def issue_gather(t, idxb, rowsb, gsem):
            pltpu.async_copy(
                table_hbm.at[idxb.at[pl.ds(0, s_a)]],
                rowsb.at[pl.ds(0, s_a)], gsem)
            pltpu.async_copy(
                table_hbm.at[idxb.at[pl.ds(s_a, s_b)]],
                rowsb.at[pl.ds(s_a, s_b)], gsem)

        bufs = ((idx0, rows0, outs0, gs0, is0, os0),
                (idx1, rows1, outs1, gs1, is1, os1))

        for b in range(2):
            idxb, rowsb, _, gsem, _, _ = bufs[b]
            base = (first + b) * seq
            pltpu.sync_copy(idx_hbm.at[pl.ds(base, seq)], idxb)
            issue_gather(b, idxb, rowsb, gsem)

        def process(t, buf):
            idxb, rowsb, outb, gsem, isem, osem = buf
            pltpu.make_async_copy(
                table_hbm.at[idxb], rowsb, gsem).wait()
            @pl.when(t + 2 < n_chunks)
            def _():
                base2 = (first + t + 2) * seq
                pltpu.async_copy(idx_hbm.at[pl.ds(base2, seq)], idxb, isem)
            @pl.when(t >= 2)
            def _():
                pltpu.make_async_copy(
                    outb, out_hbm.at[pl.ds(0, seq)], osem).wait()

            @plsc.parallel_loop(0, seq, 1, unroll=8)
            def _(i):
                for c in range(dim // 16):
                    sl = pl.ds(c * 16, 16)
                    outb[i, sl] = rowsb[i, sl] + pe_v[i, sl]

            base = (first + t) * seq
            pltpu.async_copy(outb, out_hbm.at[pl.ds(base, seq)], osem)
            @pl.when(t + 2 < n_chunks)
            def _():
                pltpu.make_async_copy(
                    idx_hbm.at[pl.ds(0, seq)], idxb, isem).wait()
                issue_gather(t + 2, idxb, rowsb, gsem)

        def step(g, carry):
            process(2 * g, bufs[0])
            process(2 * g + 1, bufs[1])
            return carry

        lax.fori_loop(0, n_chunks // 2, step, 0)

        for b in range(2):
            _, _, outb, _, _, osem = bufs[b]
            pltpu.make_async_copy(outb, out_hbm.at[pl.ds(0, seq)], osem).wait()

    return body(idx, pe, staged)


def kernel(x, table):
    batch, seq = x.shape
    dim = table.shape[1]
    idx = x.reshape(-1).astype(jnp.int32)
    pe = _pos_encoding(seq, dim)
    n_full = (table.shape[0] // 128) * 128
    tailp = jnp.pad(table[n_full:, :], ((0, 0), (0, 128 - dim)))
    staged = _sc_stage(table.T, tailp)
    out = _sc_embed(idx, pe, staged, batch, seq)
    return out.reshape(batch, seq, dim)
